# baseline (speedup 1.0000x reference)
import jax
import jax.numpy as jnp
from jax.experimental import pallas as pl
from jax.experimental.pallas import tpu as pltpu


def _heads_kernel(x_ref, wc_ref, bc_ref, wb_ref, bb_ref, sc_ref, pd_ref):
    x = x_ref[...]
    sc_ref[...] = jnp.dot(x, wc_ref[...],
                          preferred_element_type=jnp.float32) + bc_ref[...]
    pd_ref[...] = jnp.dot(x, wb_ref[...],
                          preferred_element_type=jnp.float32) + bb_ref[...]


def kernel(x, W_cls, b_cls, W_box, b_box):
    if x.ndim > 2:
        x = x.reshape(x.shape[0], -1)
    n, d = x.shape
    kc = W_cls.shape[1]
    kb = W_box.shape[1]
    bm = 2000
    assert n % bm == 0
    bc2 = b_cls.reshape(1, kc)
    bb2 = b_box.reshape(1, kb)

    scores, deltas = pl.pallas_call(
        _heads_kernel,
        grid=(n // bm,),
        in_specs=[
            pl.BlockSpec((bm, d), lambda i: (i, 0)),
            pl.BlockSpec((d, kc), lambda i: (0, 0)),
            pl.BlockSpec((1, kc), lambda i: (0, 0)),
            pl.BlockSpec((d, kb), lambda i: (0, 0)),
            pl.BlockSpec((1, kb), lambda i: (0, 0)),
        ],
        out_specs=[
            pl.BlockSpec((bm, kc), lambda i: (i, 0)),
            pl.BlockSpec((bm, kb), lambda i: (i, 0)),
        ],
        out_shape=[
            jax.ShapeDtypeStruct((n, kc), jnp.float32),
            jax.ShapeDtypeStruct((n, kb), jnp.float32),
        ],
        compiler_params=pltpu.CompilerParams(
            dimension_semantics=("parallel",),
        ),
    )(x, W_cls, bc2, W_box, bb2)
    return (scores, deltas)


# speedup vs baseline: 1.0016x; 1.0016x over previous
import jax
import jax.numpy as jnp
from jax.experimental import pallas as pl
from jax.experimental.pallas import tpu as pltpu


def _heads_kernel(x_ref, wc_ref, bc_ref, wb_ref, bb_ref, sc_ref, pd_ref):
    x = x_ref[...].astype(jnp.bfloat16)
    wc = wc_ref[...].astype(jnp.bfloat16)
    wb = wb_ref[...].astype(jnp.bfloat16)
    sc_ref[...] = jnp.dot(x, wc, preferred_element_type=jnp.float32) + bc_ref[...]
    pd_ref[...] = jnp.dot(x, wb, preferred_element_type=jnp.float32) + bb_ref[...]


def kernel(x, W_cls, b_cls, W_box, b_box):
    if x.ndim > 2:
        x = x.reshape(x.shape[0], -1)
    n, d = x.shape
    kc = W_cls.shape[1]
    kb = W_box.shape[1]
    bm = 2000
    assert n % bm == 0
    bc2 = b_cls.reshape(1, kc)
    bb2 = b_box.reshape(1, kb)

    scores, deltas = pl.pallas_call(
        _heads_kernel,
        grid=(n // bm,),
        in_specs=[
            pl.BlockSpec((bm, d), lambda i: (i, 0)),
            pl.BlockSpec((d, kc), lambda i: (0, 0)),
            pl.BlockSpec((1, kc), lambda i: (0, 0)),
            pl.BlockSpec((d, kb), lambda i: (0, 0)),
            pl.BlockSpec((1, kb), lambda i: (0, 0)),
        ],
        out_specs=[
            pl.BlockSpec((bm, kc), lambda i: (i, 0)),
            pl.BlockSpec((bm, kb), lambda i: (i, 0)),
        ],
        out_shape=[
            jax.ShapeDtypeStruct((n, kc), jnp.float32),
            jax.ShapeDtypeStruct((n, kb), jnp.float32),
        ],
        compiler_params=pltpu.CompilerParams(
            dimension_semantics=("parallel",),
        ),
    )(x, W_cls, bc2, W_box, bb2)
    return (scores, deltas)


# speedup vs baseline: 1.0021x; 1.0005x over previous
import jax
import jax.numpy as jnp
from jax.experimental import pallas as pl
from jax.experimental.pallas import tpu as pltpu


def _heads_kernel(x_ref, wc_ref, bc_ref, wb_ref, bb_ref, sc_ref, pd_ref):
    x = x_ref[...].astype(jnp.bfloat16)
    wc = wc_ref[...].astype(jnp.bfloat16)
    wb = wb_ref[...].astype(jnp.bfloat16)
    sc_ref[...] = jax.lax.dot_general(
        x, wc, (((1,), (0,)), ((), ())),
        precision=jax.lax.Precision.DEFAULT,
        preferred_element_type=jnp.float32) + bc_ref[...]
    pd_ref[...] = jax.lax.dot_general(
        x, wb, (((1,), (0,)), ((), ())),
        precision=jax.lax.Precision.DEFAULT,
        preferred_element_type=jnp.float32) + bb_ref[...]


def kernel(x, W_cls, b_cls, W_box, b_box):
    if x.ndim > 2:
        x = x.reshape(x.shape[0], -1)
    n, d = x.shape
    kc = W_cls.shape[1]
    kb = W_box.shape[1]
    bm = 2000
    assert n % bm == 0
    bc2 = b_cls.reshape(1, kc)
    bb2 = b_box.reshape(1, kb)

    scores, deltas = pl.pallas_call(
        _heads_kernel,
        grid=(n // bm,),
        in_specs=[
            pl.BlockSpec((bm, d), lambda i: (i, 0)),
            pl.BlockSpec((d, kc), lambda i: (0, 0)),
            pl.BlockSpec((1, kc), lambda i: (0, 0)),
            pl.BlockSpec((d, kb), lambda i: (0, 0)),
            pl.BlockSpec((1, kb), lambda i: (0, 0)),
        ],
        out_specs=[
            pl.BlockSpec((bm, kc), lambda i: (i, 0)),
            pl.BlockSpec((bm, kb), lambda i: (i, 0)),
        ],
        out_shape=[
            jax.ShapeDtypeStruct((n, kc), jnp.float32),
            jax.ShapeDtypeStruct((n, kb), jnp.float32),
        ],
        compiler_params=pltpu.CompilerParams(
            dimension_semantics=("parallel",),
        ),
    )(x, W_cls, bc2, W_box, bb2)
    return (scores, deltas)


# speedup vs baseline: 1.0219x; 1.0197x over previous
import jax
import jax.numpy as jnp
from jax.experimental import pallas as pl
from jax.experimental.pallas import tpu as pltpu

_CLS_PAD = 128


def _heads_kernel(x_ref, w_ref, bc_ref, bb_ref, sc_ref, pd_ref):
    kc = sc_ref.shape[1]
    y = jnp.dot(x_ref[...], w_ref[...], preferred_element_type=jnp.float32)
    sc_ref[...] = y[:, :kc] + bc_ref[...]
    pd_ref[...] = y[:, _CLS_PAD:] + bb_ref[...]


def kernel(x, W_cls, b_cls, W_box, b_box):
    if x.ndim > 2:
        x = x.reshape(x.shape[0], -1)
    n, d = x.shape
    kc = W_cls.shape[1]
    kb = W_box.shape[1]
    bm = 2000
    assert n % bm == 0 and kc <= _CLS_PAD

    w_cat = jnp.concatenate(
        [jnp.pad(W_cls, ((0, 0), (0, _CLS_PAD - kc))), W_box], axis=1)
    bc2 = b_cls.reshape(1, kc)
    bb2 = b_box.reshape(1, kb)

    scores, deltas = pl.pallas_call(
        _heads_kernel,
        grid=(n // bm,),
        in_specs=[
            pl.BlockSpec((bm, d), lambda i: (i, 0)),
            pl.BlockSpec((d, _CLS_PAD + kb), lambda i: (0, 0)),
            pl.BlockSpec((1, kc), lambda i: (0, 0)),
            pl.BlockSpec((1, kb), lambda i: (0, 0)),
        ],
        out_specs=[
            pl.BlockSpec((bm, kc), lambda i: (i, 0)),
            pl.BlockSpec((bm, kb), lambda i: (i, 0)),
        ],
        out_shape=[
            jax.ShapeDtypeStruct((n, kc), jnp.float32),
            jax.ShapeDtypeStruct((n, kb), jnp.float32),
        ],
        compiler_params=pltpu.CompilerParams(
            dimension_semantics=("parallel",),
        ),
    )(x, w_cat, bc2, bb2)
    return (scores, deltas)
